# Initial kernel scaffold; baseline (speedup 1.0000x reference)
import jax
import jax.numpy as jnp
from jax.experimental import pallas as pl


def kernel(loc_data, conf_data, priors, targets):
    raise NotImplementedError("write your pallas kernel here")


# speedup vs baseline: 19.7530x; 19.7530x over previous
import functools

import jax
import jax.numpy as jnp
from jax import lax
from jax.experimental import pallas as pl

_NUM_CLASSES = 21
_THRESHOLD = 0.5
_NEGPOS_RATIO = 3
_V0 = 0.1
_V1 = 0.2


def _smooth_l1(x):
    ax = jnp.abs(x)
    return jnp.where(ax < 1.0, 0.5 * ax * ax, ax - 0.5)


def _mbl_body(loc_ref, conf_ref, pri_ref, tgt_ref, out_ref):
    O = 20
    loc = loc_ref[0]
    conf = conf_ref[0]
    pri = pri_ref[...]
    tgt = tgt_ref[0]
    P = loc.shape[-1]
    C = conf.shape[0]

    f32 = jnp.float32
    tx1 = tgt[:, 0:1]
    ty1 = tgt[:, 1:2]
    tx2 = tgt[:, 2:3]
    ty2 = tgt[:, 3:4]
    lab = tgt[:, 4:5]
    pcx = pri[0:1]
    pcy = pri[1:2]
    pw = pri[2:3]
    ph = pri[3:4]

    px1 = pcx - pw / 2.0
    py1 = pcy - ph / 2.0
    px2 = pcx + pw / 2.0
    py2 = pcy + ph / 2.0

    iw = jnp.clip(jnp.minimum(tx2, px2) - jnp.maximum(tx1, px1), 0.0)
    ih = jnp.clip(jnp.minimum(ty2, py2) - jnp.maximum(ty1, py1), 0.0)
    inter = iw * ih
    area_t = jnp.clip(tx2 - tx1, 0.0) * jnp.clip(ty2 - ty1, 0.0)
    area_p = jnp.clip(px2 - px1, 0.0) * jnp.clip(py2 - py1, 0.0)
    ov = inter / jnp.maximum(area_t + area_p - inter, 1e-10)

    o_iota = lax.broadcasted_iota(jnp.int32, (O, P), 0)
    p_iota = lax.broadcasted_iota(jnp.int32, (O, P), 1)

    mx1 = jnp.max(ov, axis=0, keepdims=True)
    i1 = jnp.min(jnp.where(ov == mx1, o_iota, O), axis=0, keepdims=True)
    ov2 = jnp.where(o_iota == i1, -1.0, ov)
    mx2 = jnp.max(ov2, axis=0, keepdims=True)
    i2 = jnp.min(jnp.where(ov2 == mx2, o_iota, O), axis=0, keepdims=True)
    ov3 = jnp.where(o_iota == i2, -1.0, ov2)
    mx3 = jnp.max(ov3, axis=0, keepdims=True)
    i3 = jnp.min(jnp.where(ov3 == mx3, o_iota, O), axis=0, keepdims=True)

    mxr = jnp.max(ov, axis=1, keepdims=True)
    bpi = jnp.min(jnp.where(ov == mxr, p_iota, P), axis=1, keepdims=True)
    eqb = p_iota == bpi
    eqf = eqb.astype(f32)

    fi = jnp.max(jnp.where(eqb, o_iota, -1), axis=0, keepdims=True)
    forced = fi >= 0
    bti = jnp.where(forced, fi, i1)
    bto = jnp.where(forced, 2.0, mx1)

    def gather_truth(idx):
        oh = (o_iota == idx).astype(f32)
        g = lambda col: jnp.sum(oh * col, axis=0, keepdims=True)
        return g(tx1), g(ty1), g(tx2), g(ty2), oh

    m_x1, m_y1, m_x2, m_y2, oh1 = gather_truth(bti)
    labg = jnp.sum(oh1 * lab, axis=0, keepdims=True)
    conf_t = jnp.where(bto < _THRESHOLD, 0, labg.astype(jnp.int32))
    posb = conf_t > 0
    posf = posb.astype(f32)
    num_pos_i = jnp.sum(posb.astype(jnp.int32))
    num_pos_f = num_pos_i.astype(f32)

    def encode4(ex1, ey1, ex2, ey2):
        g0 = ((ex1 + ex2) / 2.0 - pcx) / (_V0 * pw)
        g1 = ((ey1 + ey2) / 2.0 - pcy) / (_V0 * ph)
        g2 = jnp.log(jnp.maximum((ex2 - ex1) / pw, 1e-10)) / _V1
        g3 = jnp.log(jnp.maximum((ey2 - ey1) / ph, 1e-10)) / _V1
        return g0, g1, g2, g3

    t0, t1, t2, t3 = encode4(m_x1, m_y1, m_x2, m_y2)

    l0 = loc[0:1]
    l1 = loc[1:2]
    l2 = loc[2:3]
    l3 = loc[3:4]

    sl = (_smooth_l1(l0 - t0) + _smooth_l1(l1 - t1)
          + _smooth_l1(l2 - t2) + _smooth_l1(l3 - t3))
    loss_l_b = jnp.sum(sl * posf)

    def decode4(a0, a1, a2, a3):
        dcx = pcx + (a0 * _V0) * pw
        dcy = pcy + (a1 * _V0) * ph
        dw = pw * jnp.exp(a2 * _V1)
        dh = ph * jnp.exp(a3 * _V1)
        return dcx - dw / 2.0, dcy - dh / 2.0, dcx + dw / 2.0, dcy + dh / 2.0

    dx1, dy1, dx2, dy2 = decode4(l0, l1, l2, l3)

    dn = (((1,), (1,)), ((), ()))

    def colrow(v):
        c = lax.dot_general(eqf, v, dn, preferred_element_type=f32,
                            precision=lax.Precision.HIGHEST)
        r = lax.dot_general(v, eqf, dn, preferred_element_type=f32,
                            precision=lax.Precision.HIGHEST)
        return c, r

    bx1c, bx1r = colrow(dx1)
    by1c, by1r = colrow(dy1)
    bx2c, bx2r = colrow(dx2)
    by2c, by2r = colrow(dy2)
    ibb = (jnp.clip(jnp.minimum(bx2c, bx2r) - jnp.maximum(bx1c, bx1r), 0.0)
           * jnp.clip(jnp.minimum(by2c, by2r) - jnp.maximum(by1c, by1r), 0.0))
    area_c = jnp.clip(bx2c - bx1c, 0.0) * jnp.clip(by2c - by1c, 0.0)
    area_r = jnp.clip(bx2r - bx1r, 0.0) * jnp.clip(by2r - by1r, 0.0)
    iou_bb = ibb / jnp.maximum(area_c + area_r - ibb, 1e-10)
    oo_a = lax.broadcasted_iota(jnp.int32, (O, O), 0)
    oo_b = lax.broadcasted_iota(jnp.int32, (O, O), 1)
    validm = (iou_bb > 0.0).astype(f32) * (oo_a != oo_b).astype(f32)
    tem_a_b = jnp.sum(-jnp.log(jnp.clip(1.0 - iou_bb, 1e-10, 1.0)) * validm)
    tem_b_b = jnp.sum(validm)

    def iog_term(idx):
        ex1, ey1, ex2, ey2, _ = gather_truth(idx)
        e0, e1, e2, e3 = encode4(ex1, ey1, ex2, ey2)
        gx1, gy1, gx2, gy2 = decode4(e0, e1, e2, e3)
        giw = jnp.clip(jnp.minimum(dx2, gx2) - jnp.maximum(dx1, gx1), 0.0)
        gih = jnp.clip(jnp.minimum(dy2, gy2) - jnp.maximum(dy1, gy1), 0.0)
        ginter = giw * gih
        garea = jnp.clip(gx2 - gx1, 0.0) * jnp.clip(gy2 - gy1, 0.0)
        iog = ginter / jnp.maximum(garea, 1e-10)
        return jnp.sum(-jnp.log(jnp.clip(1.0 - iog, 1e-10, 1.0)) * posf)

    iog_b = iog_term(i2) + iog_term(i3)

    cm = jnp.max(conf, axis=0, keepdims=True)
    lse = jnp.log(jnp.sum(jnp.exp(conf - cm), axis=0, keepdims=True)) + cm
    c_iota = lax.broadcasted_iota(jnp.int32, (C, P), 0)
    gathered = jnp.sum((c_iota == conf_t).astype(f32) * conf,
                       axis=0, keepdims=True)
    cemat = lse - gathered
    pos_ce = jnp.sum(cemat * posf)

    key = jnp.where(posb, 0.0, cemat)
    kb = lax.bitcast_convert_type(key, jnp.int32)
    kneg = jnp.minimum(_NEGPOS_RATIO * num_pos_i, P - 1)

    def bitstep(i, prefix):
        cand = prefix | lax.shift_left(jnp.int32(1), jnp.int32(30) - i)
        cnt = jnp.sum((kb >= cand).astype(jnp.int32))
        return jnp.where(cnt >= kneg, cand, prefix)

    prefix = lax.fori_loop(0, 31, bitstep, jnp.int32(0))
    vstar = jnp.max(jnp.where(kb == prefix, key, -1.0))
    gtmask = kb > prefix
    n_gt = jnp.sum(gtmask.astype(jnp.int32))
    negsum = (jnp.sum(jnp.where(gtmask, key, 0.0))
              + (kneg - n_gt).astype(f32) * vstar)
    loss_c_b = pos_ce + jnp.where(kneg > 0, negsum, 0.0)

    li = lax.broadcasted_iota(jnp.int32, (1, 8), 1)
    vals = ((li == 0).astype(f32) * loss_l_b
            + (li == 1).astype(f32) * iog_b
            + (li == 2).astype(f32) * tem_a_b
            + (li == 3).astype(f32) * tem_b_b
            + (li == 4).astype(f32) * loss_c_b
            + (li == 5).astype(f32) * num_pos_f)
    out_ref[0] = vals


@functools.partial(jax.jit, static_argnames=())
def kernel(loc_data, conf_data, priors, targets):
    B, P, _ = loc_data.shape
    C = conf_data.shape[-1]
    locT = jnp.transpose(loc_data, (0, 2, 1))
    confT = jnp.transpose(conf_data, (0, 2, 1))
    priT = jnp.transpose(priors, (1, 0))

    out = pl.pallas_call(
        _mbl_body,
        grid=(B,),
        in_specs=[
            pl.BlockSpec((1, 4, P), lambda b: (b, 0, 0)),
            pl.BlockSpec((1, C, P), lambda b: (b, 0, 0)),
            pl.BlockSpec((4, P), lambda b: (0, 0)),
            pl.BlockSpec((1, 20, 5), lambda b: (b, 0, 0)),
        ],
        out_specs=pl.BlockSpec((1, 1, 8), lambda b: (b, 0, 0)),
        out_shape=jax.ShapeDtypeStruct((B, 1, 8), jnp.float32),
    )(locT, confT, priT, targets)

    s = jnp.sum(out.reshape(B, 8), axis=0)
    N = s[5]
    loss_l = s[0] / N
    loss_l_repul = s[1] / N + s[2] / (s[3] + 1e-10)
    loss_c = s[4] / N
    return (loss_l, loss_l_repul, loss_c)


# speedup vs baseline: 26.5429x; 1.3437x over previous
import jax
import jax.numpy as jnp
from jax import lax
from jax.experimental import pallas as pl
from jax.experimental.pallas import tpu as pltpu

_C = 21
_THRESHOLD = 0.5
_NEGPOS_RATIO = 3
_V0 = 0.1
_V1 = 0.2
_P = 8732
_SUB = 8
_LAN = 1092
_PP = _SUB * _LAN
_O = 20


def _smooth_l1(x):
    ax = jnp.abs(x)
    return jnp.where(ax < 1.0, 0.5 * ax * ax, ax - 0.5)


def _match_body(loc_ref, conf_ref, pri_ref, tgt_ref, part_ref, key_ref):
    f32 = jnp.float32
    loc = loc_ref[0]
    pri = pri_ref[...]
    tgt = tgt_ref[0]

    pcx = pri[0]
    pcy = pri[1]
    pw = pri[2]
    ph = pri[3]
    px1 = pcx - pw / 2.0
    py1 = pcy - ph / 2.0
    px2 = pcx + pw / 2.0
    py2 = pcy + ph / 2.0
    area_p = jnp.clip(px2 - px1, 0.0) * jnp.clip(py2 - py1, 0.0)

    s_iota = lax.broadcasted_iota(jnp.int32, (_SUB, _LAN), 0)
    l_iota = lax.broadcasted_iota(jnp.int32, (_SUB, _LAN), 1)
    pidx = s_iota * _LAN + l_iota
    valid = pidx < _P

    tx1 = [tgt[o, 0] for o in range(_O)]
    ty1 = [tgt[o, 1] for o in range(_O)]
    tx2 = [tgt[o, 2] for o in range(_O)]
    ty2 = [tgt[o, 3] for o in range(_O)]
    lab = [tgt[o, 4] for o in range(_O)]

    ov = []
    mx1 = None
    i1 = None
    fi = jnp.full((_SUB, _LAN), -1, jnp.int32)
    bp_s = []
    bp_l = []
    for o in range(_O):
        iw = jnp.clip(jnp.minimum(tx2[o], px2) - jnp.maximum(tx1[o], px1), 0.0)
        ih = jnp.clip(jnp.minimum(ty2[o], py2) - jnp.maximum(ty1[o], py1), 0.0)
        inter = iw * ih
        area_t = (jnp.clip(tx2[o] - tx1[o], 0.0)
                  * jnp.clip(ty2[o] - ty1[o], 0.0))
        ov_o = inter / jnp.maximum(area_t + area_p - inter, 1e-10)
        ov_o = jnp.where(valid, ov_o, -1.0)
        ov.append(ov_o)
        if o == 0:
            mx1 = ov_o
            i1 = jnp.zeros((_SUB, _LAN), jnp.int32)
        else:
            gt = ov_o > mx1
            i1 = jnp.where(gt, o, i1)
            mx1 = jnp.maximum(mx1, ov_o)
        mxr_o = jnp.max(ov_o)
        bpi_o = jnp.min(jnp.where(ov_o == mxr_o, pidx, _PP))
        eq_o = pidx == bpi_o
        fi = jnp.where(eq_o, o, fi)
        bp_s.append(jnp.max(jnp.where(eq_o, s_iota, 0)))
        bp_l.append(jnp.max(jnp.where(eq_o, l_iota, 0)))

    def masked_argmax(excl):
        m = None
        idx = None
        for o in range(_O):
            v = jnp.where(excl(o), -2.0, ov[o])
            if o == 0:
                m = v
                idx = jnp.zeros((_SUB, _LAN), jnp.int32)
            else:
                gt = v > m
                idx = jnp.where(gt, o, idx)
                m = jnp.maximum(m, v)
        return idx

    i2 = masked_argmax(lambda o: i1 == o)
    i3 = masked_argmax(lambda o: (i1 == o) | (i2 == o))

    forced = fi >= 0
    bti = jnp.where(forced, fi, i1)
    bto = jnp.where(forced, 2.0, mx1)

    def gather(idx, cols):
        outs = [jnp.zeros((_SUB, _LAN), f32) for _ in cols]
        for o in range(_O):
            c = idx == o
            for j, col in enumerate(cols):
                outs[j] = jnp.where(c, col[o], outs[j])
        return outs

    m_x1, m_y1, m_x2, m_y2, labg = gather(bti, [tx1, ty1, tx2, ty2, lab])
    conf_t = jnp.where(bto < _THRESHOLD, 0, labg.astype(jnp.int32))
    posb = conf_t > 0
    posf = posb.astype(f32)
    num_pos_f = jnp.sum(posf)

    def encode4(ex1, ey1, ex2, ey2):
        g0 = ((ex1 + ex2) / 2.0 - pcx) / (_V0 * pw)
        g1 = ((ey1 + ey2) / 2.0 - pcy) / (_V0 * ph)
        g2 = jnp.log(jnp.maximum((ex2 - ex1) / pw, 1e-10)) / _V1
        g3 = jnp.log(jnp.maximum((ey2 - ey1) / ph, 1e-10)) / _V1
        return g0, g1, g2, g3

    def decode4(a0, a1, a2, a3):
        dcx = pcx + (a0 * _V0) * pw
        dcy = pcy + (a1 * _V0) * ph
        dw = pw * jnp.exp(a2 * _V1)
        dh = ph * jnp.exp(a3 * _V1)
        return dcx - dw / 2.0, dcy - dh / 2.0, dcx + dw / 2.0, dcy + dh / 2.0

    t0, t1, t2, t3 = encode4(m_x1, m_y1, m_x2, m_y2)
    l0 = loc[0]
    l1 = loc[1]
    l2 = loc[2]
    l3 = loc[3]
    sl = (_smooth_l1(l0 - t0) + _smooth_l1(l1 - t1)
          + _smooth_l1(l2 - t2) + _smooth_l1(l3 - t3))
    loss_l_b = jnp.sum(sl * posf)

    dx1, dy1, dx2, dy2 = decode4(l0, l1, l2, l3)

    oo_a = lax.broadcasted_iota(jnp.int32, (_O, _O), 0)
    oo_b = lax.broadcasted_iota(jnp.int32, (_O, _O), 1)
    zoo = jnp.zeros((_O, _O), f32)
    bx1c = zoo
    by1c = zoo
    bx2c = zoo
    by2c = zoo
    bx1r = zoo
    by1r = zoo
    bx2r = zoo
    by2r = zoo
    for o in range(_O):
        eq = (s_iota == bp_s[o]) & (l_iota == bp_l[o])
        vx1 = jnp.sum(jnp.where(eq, dx1, 0.0))
        vy1 = jnp.sum(jnp.where(eq, dy1, 0.0))
        vx2 = jnp.sum(jnp.where(eq, dx2, 0.0))
        vy2 = jnp.sum(jnp.where(eq, dy2, 0.0))
        ca = oo_a == o
        cb = oo_b == o
        bx1c = jnp.where(ca, vx1, bx1c)
        by1c = jnp.where(ca, vy1, by1c)
        bx2c = jnp.where(ca, vx2, bx2c)
        by2c = jnp.where(ca, vy2, by2c)
        bx1r = jnp.where(cb, vx1, bx1r)
        by1r = jnp.where(cb, vy1, by1r)
        bx2r = jnp.where(cb, vx2, bx2r)
        by2r = jnp.where(cb, vy2, by2r)
    ibb = (jnp.clip(jnp.minimum(bx2c, bx2r) - jnp.maximum(bx1c, bx1r), 0.0)
           * jnp.clip(jnp.minimum(by2c, by2r) - jnp.maximum(by1c, by1r), 0.0))
    area_c = jnp.clip(bx2c - bx1c, 0.0) * jnp.clip(by2c - by1c, 0.0)
    area_r = jnp.clip(bx2r - bx1r, 0.0) * jnp.clip(by2r - by1r, 0.0)
    iou_bb = ibb / jnp.maximum(area_c + area_r - ibb, 1e-10)
    validm = (iou_bb > 0.0).astype(f32) * (oo_a != oo_b).astype(f32)
    tem_a_b = jnp.sum(-jnp.log(jnp.clip(1.0 - iou_bb, 1e-10, 1.0)) * validm)
    tem_b_b = jnp.sum(validm)

    def iog_term(idx):
        ex1, ey1, ex2, ey2 = gather(idx, [tx1, ty1, tx2, ty2])
        gx1, gy1, gx2, gy2 = decode4(*encode4(ex1, ey1, ex2, ey2))
        giw = jnp.clip(jnp.minimum(dx2, gx2) - jnp.maximum(dx1, gx1), 0.0)
        gih = jnp.clip(jnp.minimum(dy2, gy2) - jnp.maximum(dy1, gy1), 0.0)
        ginter = giw * gih
        garea = jnp.clip(gx2 - gx1, 0.0) * jnp.clip(gy2 - gy1, 0.0)
        iog = ginter / jnp.maximum(garea, 1e-10)
        return jnp.sum(-jnp.log(jnp.clip(1.0 - iog, 1e-10, 1.0)) * posf)

    iog_b = iog_term(i2) + iog_term(i3)

    conf = conf_ref[0]
    cm = conf[0]
    for c in range(1, _C):
        cm = jnp.maximum(cm, conf[c])
    se = jnp.exp(conf[0] - cm)
    for c in range(1, _C):
        se = se + jnp.exp(conf[c] - cm)
    lse = jnp.log(se) + cm
    gathered = jnp.zeros((_SUB, _LAN), f32)
    for c in range(_C):
        gathered = jnp.where(conf_t == c, conf[c], gathered)
    cemat = lse - gathered
    pos_ce = jnp.sum(cemat * posf)

    key = jnp.where(posb | (~valid), 0.0, cemat)
    key_ref[0] = key

    li = lax.broadcasted_iota(jnp.int32, (1, 8), 1)
    vals = ((li == 0).astype(f32) * loss_l_b
            + (li == 1).astype(f32) * iog_b
            + (li == 2).astype(f32) * tem_a_b
            + (li == 3).astype(f32) * tem_b_b
            + (li == 4).astype(f32) * pos_ce
            + (li == 5).astype(f32) * num_pos_f)
    part_ref[0] = vals


def _mine_body(key_ref, part_ref, out_ref):
    f32 = jnp.float32
    keys = key_ref[...]
    kb = lax.bitcast_convert_type(keys, jnp.int32)
    npos = part_ref[...][:, 5:6].astype(jnp.int32)
    kneg = jnp.minimum(_NEGPOS_RATIO * npos, _P - 1)

    def bitstep(i, prefix):
        cand = prefix | lax.shift_left(jnp.int32(1), jnp.int32(30) - i)
        cnt = jnp.sum((kb >= cand).astype(jnp.int32), axis=1, keepdims=True)
        return jnp.where(cnt >= kneg, cand, prefix)

    prefix = lax.fori_loop(0, 31, bitstep,
                           jnp.zeros(kneg.shape, jnp.int32))
    vstar = jnp.max(jnp.where(kb == prefix, keys, -1.0),
                    axis=1, keepdims=True)
    gtm = kb > prefix
    n_gt = jnp.sum(gtm.astype(jnp.int32), axis=1, keepdims=True)
    sumgt = jnp.sum(jnp.where(gtm, keys, 0.0), axis=1, keepdims=True)
    negsum = sumgt + (kneg - n_gt).astype(f32) * vstar
    negsum = jnp.where(kneg > 0, negsum, 0.0)
    li = lax.broadcasted_iota(jnp.int32, (1, 8), 1)
    out_ref[...] = (li == 0).astype(f32) * negsum


def kernel(loc_data, conf_data, priors, targets):
    B, P, _ = loc_data.shape
    C = conf_data.shape[-1]
    pad = _PP - P
    locP = jnp.pad(jnp.transpose(loc_data, (0, 2, 1)),
                   ((0, 0), (0, 0), (0, pad))).reshape(B, 4, _SUB, _LAN)
    confP = jnp.pad(jnp.transpose(conf_data, (0, 2, 1)),
                    ((0, 0), (0, 0), (0, pad))).reshape(B, C, _SUB, _LAN)
    priP = jnp.pad(jnp.transpose(priors, (1, 0)),
                   ((0, 0), (0, pad))).reshape(4, _SUB, _LAN)

    parts, keys = pl.pallas_call(
        _match_body,
        grid=(B,),
        in_specs=[
            pl.BlockSpec((1, 4, _SUB, _LAN), lambda b: (b, 0, 0, 0)),
            pl.BlockSpec((1, C, _SUB, _LAN), lambda b: (b, 0, 0, 0)),
            pl.BlockSpec((4, _SUB, _LAN), lambda b: (0, 0, 0)),
            pl.BlockSpec((1, 20, 5), lambda b: (b, 0, 0)),
        ],
        out_specs=[
            pl.BlockSpec((1, 1, 8), lambda b: (b, 0, 0)),
            pl.BlockSpec((1, _SUB, _LAN), lambda b: (b, 0, 0)),
        ],
        out_shape=[
            jax.ShapeDtypeStruct((B, 1, 8), jnp.float32),
            jax.ShapeDtypeStruct((B, _SUB, _LAN), jnp.float32),
        ],
        compiler_params=pltpu.CompilerParams(
            dimension_semantics=("parallel",)),
    )(locP, confP, priP, targets)

    parts2 = parts.reshape(B, 8)
    neg = pl.pallas_call(
        _mine_body,
        in_specs=[
            pl.BlockSpec((B, _PP), lambda: (0, 0)),
            pl.BlockSpec((B, 8), lambda: (0, 0)),
        ],
        out_specs=pl.BlockSpec((B, 8), lambda: (0, 0)),
        out_shape=jax.ShapeDtypeStruct((B, 8), jnp.float32),
    )(keys.reshape(B, _PP), parts2)

    s = jnp.sum(parts2, axis=0)
    N = s[5]
    loss_l = s[0] / N
    loss_l_repul = s[1] / N + s[2] / (s[3] + 1e-10)
    loss_c = (s[4] + jnp.sum(neg[:, 0])) / N
    return (loss_l, loss_l_repul, loss_c)
